# Initial kernel scaffold; baseline (speedup 1.0000x reference)
import jax
import jax.numpy as jnp
from jax.experimental import pallas as pl


def kernel(pred_logits, pred_boxes, pos_maps, target_sizes):
    raise NotImplementedError("write your pallas kernel here")


# speedup vs baseline: 4.2624x; 4.2624x over previous
import functools

import jax
import jax.numpy as jnp
from jax.experimental import pallas as pl
from jax.experimental.pallas import tpu as pltpu

_K = 300
_NEG = -3.0e38
_BIGI = 2 ** 30


def _iota(shape, dim, dtype=jnp.int32):
    return jax.lax.broadcasted_iota(dtype, shape, dim)


def _eye128():
    return jnp.where(_iota((128, 128), 0) == _iota((128, 128), 1), 1.0, 0.0
                     ).astype(jnp.float32)


def _row_from_col(col):
    return jnp.transpose(col, (1, 0))


def _col_from_row(row):
    return jnp.transpose(row, (1, 0))


def _body(logits_ref, pmt_ref, boxes_ref, scale_ref,
          scores_ref, labels_ref, boxes_out_ref, m_ref):
    nq, nc = m_ref.shape
    nq_pad = ((nq + 127) // 128) * 128

    s = jax.nn.sigmoid(logits_ref[0])
    p = jnp.dot(s, pmt_ref[...], preferred_element_type=jnp.float32)
    m_ref[:, :] = p

    rm_col = jnp.max(p, axis=1, keepdims=True)
    rm_col = jnp.concatenate(
        [rm_col, jnp.full((nq_pad - nq, 1), _NEG, jnp.float32)], axis=0)
    rm = _row_from_col(rm_col)

    lane_q = _iota((1, nq_pad), 1)
    lane_c = _iota((1, nc), 1)
    lane_k = _iota((1, _K), 1)

    def step(k, carry):
        rm, vals, qs, cs = carry
        gv = jnp.max(rm)
        r = jnp.min(jnp.where(rm >= gv, lane_q, _BIGI))
        row = m_ref[pl.ds(r, 1), :]
        c = jnp.min(jnp.where(row >= gv, lane_c, _BIGI))
        sel = lane_k == k
        vals = jnp.where(sel, gv, vals)
        qs = jnp.where(sel, r, qs)
        cs = jnp.where(sel, c, cs)
        row = jnp.where(lane_c == c, _NEG, row)
        m_ref[pl.ds(r, 1), :] = row
        rm = jnp.where(lane_q == r, jnp.max(row), rm)
        return rm, vals, qs, cs

    vals0 = jnp.zeros((1, _K), jnp.float32)
    qs0 = jnp.zeros((1, _K), jnp.int32)
    cs0 = jnp.zeros((1, _K), jnp.int32)
    rm, vals, qs, cs = jax.lax.fori_loop(0, _K, step, (rm, vals0, qs0, cs0))

    scores_ref[0] = vals
    labels_ref[0] = cs

    b4 = boxes_ref[0]
    cx, cy, w, h = (b4[:, 0:1], b4[:, 1:2], b4[:, 2:3], b4[:, 3:4])
    cb = jnp.concatenate(
        [cx - 0.5 * w, cy - 0.5 * h, cx + 0.5 * w, cy + 0.5 * h], axis=1)

    k_pad = ((_K + 127) // 128) * 128
    qs_f = jnp.concatenate(
        [qs.astype(jnp.float32),
         jnp.full((1, k_pad - _K), -1.0, jnp.float32)], axis=1)
    qs_t = _col_from_row(qs_f)[:_K]
    q_iota = _iota((_K, nq), 1).astype(jnp.float32)
    onehot = jnp.where(qs_t == q_iota, 1.0, 0.0)
    bsel = jnp.dot(onehot, cb, preferred_element_type=jnp.float32)
    boxes_out_ref[0] = bsel * scale_ref[0]


@jax.jit
def kernel(pred_logits, pred_boxes, pos_maps, target_sizes):
    bsz, nq, txt = pred_logits.shape
    nc = pos_maps.shape[0]
    pmt = pos_maps.T
    img_h = target_sizes[:, 0]
    img_w = target_sizes[:, 1]
    scale = jnp.stack([img_w, img_h, img_w, img_h], axis=1
                      ).reshape(bsz, 1, 4)

    scores3, labels3, boxes = pl.pallas_call(
        _body,
        grid=(bsz,),
        in_specs=[
            pl.BlockSpec((1, nq, txt), lambda b: (b, 0, 0)),
            pl.BlockSpec((txt, nc), lambda b: (0, 0)),
            pl.BlockSpec((1, nq, 4), lambda b: (b, 0, 0)),
            pl.BlockSpec((1, 1, 4), lambda b: (b, 0, 0)),
        ],
        out_specs=[
            pl.BlockSpec((1, 1, _K), lambda b: (b, 0, 0)),
            pl.BlockSpec((1, 1, _K), lambda b: (b, 0, 0)),
            pl.BlockSpec((1, _K, 4), lambda b: (b, 0, 0)),
        ],
        out_shape=[
            jax.ShapeDtypeStruct((bsz, 1, _K), jnp.float32),
            jax.ShapeDtypeStruct((bsz, 1, _K), jnp.int32),
            jax.ShapeDtypeStruct((bsz, _K, 4), jnp.float32),
        ],
        scratch_shapes=[pltpu.VMEM((nq, nc), jnp.float32)],
        compiler_params=pltpu.CompilerParams(
            dimension_semantics=("arbitrary",)),
    )(pred_logits, pmt, pred_boxes, scale)

    return scores3[:, 0, :], labels3[:, 0, :], boxes


# speedup vs baseline: 4.2627x; 1.0001x over previous
import functools

import jax
import jax.numpy as jnp
from jax.experimental import pallas as pl
from jax.experimental.pallas import tpu as pltpu

_K = 300
_NEG = -3.0e38
_BIGI = 2 ** 30


def _iota(shape, dim, dtype=jnp.int32):
    return jax.lax.broadcasted_iota(dtype, shape, dim)


def _eye128():
    return jnp.where(_iota((128, 128), 0) == _iota((128, 128), 1), 1.0, 0.0
                     ).astype(jnp.float32)


def _row_from_col(col):
    return jnp.transpose(col, (1, 0))


def _col_from_row(row):
    return jnp.transpose(row, (1, 0))


def _body(logits_ref, pmt_ref, boxes_ref, scale_ref,
          scores_ref, labels_ref, boxes_out_ref, m_ref):
    nq, nc = m_ref.shape
    nq_pad = ((nq + 127) // 128) * 128

    s = jax.nn.sigmoid(logits_ref[0])
    p = jnp.dot(s, pmt_ref[...], preferred_element_type=jnp.float32)
    m_ref[:, :] = p

    rm_col = jnp.max(p, axis=1, keepdims=True)
    rm_col = jnp.concatenate(
        [rm_col, jnp.full((nq_pad - nq, 1), _NEG, jnp.float32)], axis=0)
    rm = _row_from_col(rm_col)

    lane_q = _iota((1, nq_pad), 1)
    lane_c = _iota((1, nc), 1)
    lane_k = _iota((1, _K), 1)

    def step(k, carry):
        rm, vals, qs, cs = carry
        gv = jnp.max(rm)
        r = jnp.min(jnp.where(rm >= gv, lane_q, _BIGI))
        row = m_ref[pl.ds(r, 1), :]
        c = jnp.min(jnp.where(row >= gv, lane_c, _BIGI))
        sel = lane_k == k
        vals = jnp.where(sel, gv, vals)
        qs = jnp.where(sel, r, qs)
        cs = jnp.where(sel, c, cs)
        row = jnp.where(lane_c == c, _NEG, row)
        m_ref[pl.ds(r, 1), :] = row
        rm = jnp.where(lane_q == r, jnp.max(row), rm)
        return rm, vals, qs, cs

    vals0 = jnp.zeros((1, _K), jnp.float32)
    qs0 = jnp.zeros((1, _K), jnp.int32)
    cs0 = jnp.zeros((1, _K), jnp.int32)
    rm, vals, qs, cs = jax.lax.fori_loop(0, _K, step, (rm, vals0, qs0, cs0))

    scores_ref[0] = vals
    labels_ref[0] = cs

    b4 = boxes_ref[0]
    cx, cy, w, h = (b4[:, 0:1], b4[:, 1:2], b4[:, 2:3], b4[:, 3:4])
    cb = jnp.concatenate(
        [cx - 0.5 * w, cy - 0.5 * h, cx + 0.5 * w, cy + 0.5 * h], axis=1)

    k_pad = ((_K + 127) // 128) * 128
    qs_f = jnp.concatenate(
        [qs.astype(jnp.float32),
         jnp.full((1, k_pad - _K), -1.0, jnp.float32)], axis=1)
    qs_t = _col_from_row(qs_f)[:_K]
    q_iota = _iota((_K, nq), 1).astype(jnp.float32)
    onehot = jnp.where(qs_t == q_iota, 1.0, 0.0)
    bsel = jnp.dot(onehot, cb, preferred_element_type=jnp.float32)
    boxes_out_ref[0] = bsel * scale_ref[0]


@jax.jit
def kernel(pred_logits, pred_boxes, pos_maps, target_sizes):
    bsz, nq, txt = pred_logits.shape
    nc = pos_maps.shape[0]
    pmt = pos_maps.T
    img_h = target_sizes[:, 0]
    img_w = target_sizes[:, 1]
    scale = jnp.stack([img_w, img_h, img_w, img_h], axis=1
                      ).reshape(bsz, 1, 4)

    scores3, labels3, boxes = pl.pallas_call(
        _body,
        grid=(bsz,),
        in_specs=[
            pl.BlockSpec((1, nq, txt), lambda b: (b, 0, 0)),
            pl.BlockSpec((txt, nc), lambda b: (0, 0)),
            pl.BlockSpec((1, nq, 4), lambda b: (b, 0, 0)),
            pl.BlockSpec((1, 1, 4), lambda b: (b, 0, 0)),
        ],
        out_specs=[
            pl.BlockSpec((1, 1, _K), lambda b: (b, 0, 0)),
            pl.BlockSpec((1, 1, _K), lambda b: (b, 0, 0)),
            pl.BlockSpec((1, _K, 4), lambda b: (b, 0, 0)),
        ],
        out_shape=[
            jax.ShapeDtypeStruct((bsz, 1, _K), jnp.float32),
            jax.ShapeDtypeStruct((bsz, 1, _K), jnp.int32),
            jax.ShapeDtypeStruct((bsz, _K, 4), jnp.float32),
        ],
        scratch_shapes=[pltpu.VMEM((nq, nc), jnp.float32)],
        compiler_params=pltpu.CompilerParams(
            dimension_semantics=("parallel",)),
    )(pred_logits, pmt, pred_boxes, scale)

    return scores3[:, 0, :], labels3[:, 0, :], boxes


# speedup vs baseline: 4.6994x; 1.1025x over previous
import jax
import jax.numpy as jnp
from jax.experimental import pallas as pl
from jax.experimental.pallas import tpu as pltpu

_K = 300
_G = 2
_NEG = -3.0e38
_BIGI = 2 ** 30


def _iota(shape, dim, dtype=jnp.int32):
    return jax.lax.broadcasted_iota(dtype, shape, dim)


def _body(logits_ref, pmt_ref, boxes_ref, scale_ref,
          scores_ref, labels_ref, boxes_out_ref, m_ref):
    _, nq, nc = m_ref.shape
    nq_pad = ((nq + 127) // 128) * 128

    lane_q = _iota((1, nq_pad), 1)
    lane_c = _iota((1, nc), 1)
    lane_k = _iota((1, _K), 1)

    rms, vals0, qs0, cs0 = [], [], [], []
    for i in range(_G):
        s = jax.nn.sigmoid(logits_ref[i])
        p = jnp.dot(s, pmt_ref[...], preferred_element_type=jnp.float32)
        m_ref[i] = p
        rm_col = jnp.max(p, axis=1, keepdims=True)
        rm_col = jnp.concatenate(
            [rm_col, jnp.full((nq_pad - nq, 1), _NEG, jnp.float32)], axis=0)
        rms.append(jnp.transpose(rm_col, (1, 0)))
        vals0.append(jnp.zeros((1, _K), jnp.float32))
        qs0.append(jnp.zeros((1, _K), jnp.int32))
        cs0.append(jnp.zeros((1, _K), jnp.int32))

    def step(k, carry):
        rms, vals, qs, cs = map(list, carry)
        sel = lane_k == k
        for i in range(_G):
            gv = jnp.max(rms[i])
            r = jnp.min(jnp.where(rms[i] >= gv, lane_q, _BIGI))
            row = m_ref[i, pl.ds(r, 1), :]
            c = jnp.min(jnp.where(row >= gv, lane_c, _BIGI))
            vals[i] = jnp.where(sel, gv, vals[i])
            qs[i] = jnp.where(sel, r, qs[i])
            cs[i] = jnp.where(sel, c, cs[i])
            row = jnp.where(lane_c == c, _NEG, row)
            m_ref[i, pl.ds(r, 1), :] = row
            rms[i] = jnp.where(lane_q == r, jnp.max(row), rms[i])
        return tuple(rms), tuple(vals), tuple(qs), tuple(cs)

    carry = (tuple(rms), tuple(vals0), tuple(qs0), tuple(cs0))
    rms, vals, qs, cs = jax.lax.fori_loop(0, _K, step, carry)

    q_iota = _iota((_K, nq), 1).astype(jnp.float32)
    for i in range(_G):
        scores_ref[i] = vals[i]
        labels_ref[i] = cs[i]

        b4 = boxes_ref[i]
        cx, cy, w, h = (b4[:, 0:1], b4[:, 1:2], b4[:, 2:3], b4[:, 3:4])
        cb = jnp.concatenate(
            [cx - 0.5 * w, cy - 0.5 * h, cx + 0.5 * w, cy + 0.5 * h], axis=1)
        k_pad = ((_K + 127) // 128) * 128
        qs_f = jnp.concatenate(
            [qs[i].astype(jnp.float32),
             jnp.full((1, k_pad - _K), -1.0, jnp.float32)], axis=1)
        qs_t = jnp.transpose(qs_f, (1, 0))[:_K]
        onehot = jnp.where(qs_t == q_iota, 1.0, 0.0)
        bsel = jnp.dot(onehot, cb, preferred_element_type=jnp.float32)
        boxes_out_ref[i] = bsel * scale_ref[i]


@jax.jit
def kernel(pred_logits, pred_boxes, pos_maps, target_sizes):
    bsz, nq, txt = pred_logits.shape
    nc = pos_maps.shape[0]
    pmt = pos_maps.T
    img_h = target_sizes[:, 0]
    img_w = target_sizes[:, 1]
    scale = jnp.stack([img_w, img_h, img_w, img_h], axis=1
                      ).reshape(bsz, 1, 4)

    scores3, labels3, boxes = pl.pallas_call(
        _body,
        grid=(bsz // _G,),
        in_specs=[
            pl.BlockSpec((_G, nq, txt), lambda b: (b, 0, 0)),
            pl.BlockSpec((txt, nc), lambda b: (0, 0)),
            pl.BlockSpec((_G, nq, 4), lambda b: (b, 0, 0)),
            pl.BlockSpec((_G, 1, 4), lambda b: (b, 0, 0)),
        ],
        out_specs=[
            pl.BlockSpec((_G, 1, _K), lambda b: (b, 0, 0)),
            pl.BlockSpec((_G, 1, _K), lambda b: (b, 0, 0)),
            pl.BlockSpec((_G, _K, 4), lambda b: (b, 0, 0)),
        ],
        out_shape=[
            jax.ShapeDtypeStruct((bsz, 1, _K), jnp.float32),
            jax.ShapeDtypeStruct((bsz, 1, _K), jnp.int32),
            jax.ShapeDtypeStruct((bsz, _K, 4), jnp.float32),
        ],
        scratch_shapes=[pltpu.VMEM((_G, nq, nc), jnp.float32)],
        compiler_params=pltpu.CompilerParams(
            dimension_semantics=("parallel",)),
    )(pred_logits, pmt, pred_boxes, scale)

    return scores3[:, 0, :], labels3[:, 0, :], boxes
